# baseline (speedup 1.0000x reference)
import jax
import jax.numpy as jnp
from jax import lax
from jax.experimental import pallas as pl
from jax.experimental.pallas import tpu as pltpu
from jax.experimental.pallas import tpu_sc as plsc

_KS0 = 0
_KS1 = 12345
_KS2 = _KS0 ^ _KS1 ^ 0x1BD11BDA
_ROTS = ((13, 15, 26, 6), (17, 29, 16, 24))

_ROWS = 1024
_BLOCK = _ROWS * 128
_NNZ = 4294967
_GRID = (_NNZ + _BLOCK - 1) // _BLOCK

_IBLK = 576 * 128
_TCCOLS = _GRID * _IBLK

_NW = 32
_W = 16384
_GFULL = (_NNZ - _TCCOLS) // _W
_SCEND = _TCCOLS + _GFULL * _W
_WLAST = ((_NNZ - _SCEND) // 128) * 128
_SCEND2 = _SCEND + _WLAST
_SCROUNDS = (_GFULL + _NW - 1) // _NW


def _idx_copy_sc(xi_ref, oi_ref, buf):
    wid = lax.axis_index("s") * 2 + lax.axis_index("c")
    for k in range(_SCROUNDS):
        g = k * _NW + wid

        @pl.when(g < _GFULL)
        def _copy_chunk():
            off = pl.multiple_of(_TCCOLS + g * _W, 128)
            pltpu.sync_copy(xi_ref.at[:, pl.ds(off, _W)], buf)
            pltpu.sync_copy(buf, oi_ref.at[:, pl.ds(off, _W)])

    @pl.when(wid == _NW - 1)
    def _copy_last():
        pltpu.sync_copy(
            xi_ref.at[:, pl.ds(_SCEND, _WLAST)],
            buf.at[:, pl.ds(0, _WLAST)],
        )
        pltpu.sync_copy(
            buf.at[:, pl.ds(0, _WLAST)],
            oi_ref.at[:, pl.ds(_SCEND, _WLAST)],
        )


def _dropout_body(idx_ref, x_ref, sc_ref, oi_ref, o_ref):
    del sc_ref
    oi_ref[...] = idx_ref[...]

    base = pl.program_id(0) * _BLOCK
    idx = (
        base
        + 128 * lax.broadcasted_iota(jnp.int32, (_ROWS, 128), 0)
        + lax.broadcasted_iota(jnp.int32, (_ROWS, 128), 1)
    )
    ks = (jnp.uint32(_KS0), jnp.uint32(_KS1), jnp.uint32(_KS2))
    x0 = jnp.full((_ROWS, 128), _KS0, jnp.uint32)
    x1 = idx.astype(jnp.uint32) + ks[1]
    for i in range(5):
        for r in _ROTS[i % 2]:
            x0 = x0 + x1
            x1 = (x1 << jnp.uint32(r)) | (x1 >> jnp.uint32(32 - r))
            x1 = x1 ^ x0
        x0 = x0 + ks[(i + 1) % 3]
        x1 = x1 + ks[(i + 2) % 3] + jnp.uint32(i + 1)
    keep = (x0 ^ x1) >= jnp.uint32(0x80000000)
    x = x_ref[...].reshape(_ROWS, 128)
    out = jnp.where(keep, x * jnp.float32(2.0), jnp.float32(0.0))
    o_ref[...] = out.reshape(_BLOCK)


def kernel(x_indices, x_values):
    n = x_values.shape[0]
    oi_sc = pl.kernel(
        _idx_copy_sc,
        out_type=jax.ShapeDtypeStruct(x_indices.shape, x_indices.dtype),
        mesh=plsc.VectorSubcoreMesh(core_axis_name="c", subcore_axis_name="s"),
        scratch_types=[pltpu.VMEM((2, _W), jnp.int32)],
    )(x_indices)
    oi, out = pl.pallas_call(
        _dropout_body,
        grid=(pl.cdiv(n, _BLOCK),),
        in_specs=[
            pl.BlockSpec((2, _IBLK), lambda i: (0, i)),
            pl.BlockSpec((_BLOCK,), lambda i: (i,)),
            pl.BlockSpec(memory_space=pl.ANY),
        ],
        out_specs=[
            pl.BlockSpec((2, _IBLK), lambda i: (0, i)),
            pl.BlockSpec((_BLOCK,), lambda i: (i,)),
        ],
        out_shape=[
            jax.ShapeDtypeStruct(x_indices.shape, x_indices.dtype),
            jax.ShapeDtypeStruct((n,), jnp.float32),
        ],
        input_output_aliases={2: 0},
    )(x_indices, x_values, oi_sc)
    tail = lax.slice(x_indices, (0, _SCEND2), (2, _NNZ))
    oi = lax.dynamic_update_slice(oi, tail, (0, _SCEND2))
    return (oi, out)


# speedup vs baseline: 1.4579x; 1.4579x over previous
import jax
import jax.numpy as jnp
from jax import lax
from jax.experimental import pallas as pl

_KS0 = 0
_KS1 = 12345
_KS2 = _KS0 ^ _KS1 ^ 0x1BD11BDA
_ROTS = ((13, 15, 26, 6), (17, 29, 16, 24))

_ROWS = 1024
_BLOCK = _ROWS * 128


def _dropout_body(idx_ref, x_ref, oi_ref, o_ref):
    oi_ref[...] = idx_ref[...]

    base = pl.program_id(0) * _BLOCK
    idx = (
        base
        + 128 * lax.broadcasted_iota(jnp.int32, (_ROWS, 128), 0)
        + lax.broadcasted_iota(jnp.int32, (_ROWS, 128), 1)
    )
    ks = (jnp.uint32(_KS0), jnp.uint32(_KS1), jnp.uint32(_KS2))
    x0 = jnp.full((_ROWS, 128), _KS0, jnp.uint32)
    x1 = idx.astype(jnp.uint32) + ks[1]
    for i in range(5):
        for r in _ROTS[i % 2]:
            x0 = x0 + x1
            x1 = (x1 << jnp.uint32(r)) | (x1 >> jnp.uint32(32 - r))
            x1 = x1 ^ x0
        x0 = x0 + ks[(i + 1) % 3]
        x1 = x1 + ks[(i + 2) % 3] + jnp.uint32(i + 1)
    keep = (x0 ^ x1) >= jnp.uint32(0x80000000)
    x = x_ref[...].reshape(_ROWS, 128)
    out = jnp.where(keep, x * jnp.float32(2.0), jnp.float32(0.0))
    o_ref[...] = out.reshape(_BLOCK)


def kernel(x_indices, x_values):
    n = x_values.shape[0]
    oi, out = pl.pallas_call(
        _dropout_body,
        grid=(pl.cdiv(n, _BLOCK),),
        in_specs=[
            pl.BlockSpec((2, _BLOCK), lambda i: (0, i)),
            pl.BlockSpec((_BLOCK,), lambda i: (i,)),
        ],
        out_specs=[
            pl.BlockSpec((2, _BLOCK), lambda i: (0, i)),
            pl.BlockSpec((_BLOCK,), lambda i: (i,)),
        ],
        out_shape=[
            jax.ShapeDtypeStruct(x_indices.shape, x_indices.dtype),
            jax.ShapeDtypeStruct((n,), jnp.float32),
        ],
    )(x_indices, x_values)
    return (oi, out)
